# Initial kernel scaffold; baseline (speedup 1.0000x reference)
import jax
import jax.numpy as jnp
from jax.experimental import pallas as pl


def kernel(xyz1, xyz2, points1, points2, W1, b1, g1, be1, W2, b2, g2, be2):
    raise NotImplementedError("write your pallas kernel here")


# speedup vs baseline: 9.9333x; 9.9333x over previous
import functools

import jax
import jax.numpy as jnp
from jax.experimental import pallas as pl
from jax.experimental.pallas import tpu as pltpu

_HI = jax.lax.Precision.HIGHEST


def _knn_interp_body(nsrc, x1_ref, x2_ref, p2_ref, out_ref):
    a = x1_ref[0]
    c = x2_ref[0]
    na = jnp.sum(a * a, axis=1, keepdims=True)
    nc = jnp.sum(c * c, axis=0, keepdims=True)
    ab = a.astype(jnp.bfloat16).astype(jnp.float32)
    cb = c.astype(jnp.bfloat16).astype(jnp.float32)
    cross = (ab[:, 0:1] * cb[0:1, :]
             + ab[:, 1:2] * cb[1:2, :]
             + ab[:, 2:3] * cb[2:3, :])
    d2 = jnp.maximum(na + nc - 2.0 * cross, 0.0)
    lane = jax.lax.broadcasted_iota(jnp.int32, d2.shape, 1)
    dcur = d2
    sels = []
    dks = []
    for _ in range(3):
        m = jnp.min(dcur, axis=1, keepdims=True)
        ik = jnp.min(jnp.where(dcur == m, lane, nsrc), axis=1, keepdims=True)
        sel = lane == ik
        dks.append(jnp.sqrt(m))
        dcur = jnp.where(sel, jnp.float32(jnp.inf), dcur)
        sels.append(sel)
    ws = [1.0 / (dk + 1e-10) for dk in dks]
    wsum = ws[0] + ws[1] + ws[2]
    wsp = ((ws[0] / wsum) * sels[0]
           + (ws[1] / wsum) * sels[1]
           + (ws[2] / wsum) * sels[2])
    out_ref[0] = jnp.dot(wsp, p2_ref[0], preferred_element_type=jnp.float32,
                         precision=_HI)


def _mlp1_body(x1_ref, x2_ref, w1a_ref, w1b_ref, b1_ref, y_ref, s_ref, q_ref):
    i = pl.program_id(0)
    y = (jnp.dot(x1_ref[...], w1a_ref[...], preferred_element_type=jnp.float32,
                 precision=_HI)
         + jnp.dot(x2_ref[...], w1b_ref[...], preferred_element_type=jnp.float32,
                   precision=_HI)
         + b1_ref[...])
    y_ref[...] = y

    @pl.when(i == 0)
    def _():
        s_ref[...] = jnp.zeros_like(s_ref)
        q_ref[...] = jnp.zeros_like(q_ref)

    s_ref[...] += jnp.sum(y, axis=0, keepdims=True)
    q_ref[...] += jnp.sum(y * y, axis=0, keepdims=True)


def _mlp2_body(count, y1_ref, s1_ref, q1_ref, g1_ref, be1_ref, w2_ref, b2_ref,
               y2_ref, s_ref, q_ref):
    i = pl.program_id(0)
    inv_n = jnp.float32(1.0 / count)
    mean = s1_ref[...] * inv_n
    var = q1_ref[...] * inv_n - mean * mean
    scale = g1_ref[...] / jnp.sqrt(var + 1e-5)
    h = jnp.maximum((y1_ref[...] - mean) * scale + be1_ref[...], 0.0)
    y2 = jnp.dot(h, w2_ref[...], preferred_element_type=jnp.float32,
                 precision=_HI) + b2_ref[...]
    y2_ref[...] = y2

    @pl.when(i == 0)
    def _():
        s_ref[...] = jnp.zeros_like(s_ref)
        q_ref[...] = jnp.zeros_like(q_ref)

    s_ref[...] += jnp.sum(y2, axis=0, keepdims=True)
    q_ref[...] += jnp.sum(y2 * y2, axis=0, keepdims=True)


def _final_body(count, y2_ref, s2_ref, q2_ref, g2_ref, be2_ref, out_ref):
    inv_n = jnp.float32(1.0 / count)
    mean = s2_ref[...] * inv_n
    var = q2_ref[...] * inv_n - mean * mean
    scale = g2_ref[...] / jnp.sqrt(var + 1e-5)
    out_ref[...] = jnp.maximum((y2_ref[...] - mean) * scale + be2_ref[...], 0.0)


def kernel(xyz1, xyz2, points1, points2, W1, b1, g1, be1, W2, b2, g2, be2):
    B, _, N = xyz1.shape
    S = xyz2.shape[2]
    D1 = points1.shape[1]
    D2 = points2.shape[1]
    C1 = W1.shape[0]
    C2 = W2.shape[0]

    NB = min(256, N)
    x1t = jnp.transpose(xyz1, (0, 2, 1))
    p2t = jnp.transpose(points2, (0, 2, 1))

    interp = pl.pallas_call(
        functools.partial(_knn_interp_body, S),
        grid=(B, N // NB),
        in_specs=[
            pl.BlockSpec((1, NB, 3), lambda b, i: (b, i, 0)),
            pl.BlockSpec((1, 3, S), lambda b, i: (b, 0, 0)),
            pl.BlockSpec((1, S, D2), lambda b, i: (b, 0, 0)),
        ],
        out_specs=pl.BlockSpec((1, NB, D2), lambda b, i: (b, i, 0)),
        out_shape=jax.ShapeDtypeStruct((B, N, D2), jnp.float32),
    )(x1t, xyz2, p2t)

    R = B * N
    M = min(512, R)
    G = R // M
    p1t = jnp.transpose(points1, (0, 2, 1)).reshape(R, D1)
    x2 = interp.reshape(R, D2)
    w1aT = jnp.transpose(W1[:, :D1])
    w1bT = jnp.transpose(W1[:, D1:])
    w2T = jnp.transpose(W2)

    row2 = lambda v: v.reshape(1, -1)

    y1, s1, q1 = pl.pallas_call(
        _mlp1_body,
        grid=(G,),
        in_specs=[
            pl.BlockSpec((M, D1), lambda i: (i, 0)),
            pl.BlockSpec((M, D2), lambda i: (i, 0)),
            pl.BlockSpec((D1, C1), lambda i: (0, 0)),
            pl.BlockSpec((D2, C1), lambda i: (0, 0)),
            pl.BlockSpec((1, C1), lambda i: (0, 0)),
        ],
        out_specs=[
            pl.BlockSpec((M, C1), lambda i: (i, 0)),
            pl.BlockSpec((1, C1), lambda i: (0, 0)),
            pl.BlockSpec((1, C1), lambda i: (0, 0)),
        ],
        out_shape=[
            jax.ShapeDtypeStruct((R, C1), jnp.float32),
            jax.ShapeDtypeStruct((1, C1), jnp.float32),
            jax.ShapeDtypeStruct((1, C1), jnp.float32),
        ],
    )(p1t, x2, w1aT, w1bT, row2(b1))

    y2, s2, q2 = pl.pallas_call(
        functools.partial(_mlp2_body, R),
        grid=(G,),
        in_specs=[
            pl.BlockSpec((M, C1), lambda i: (i, 0)),
            pl.BlockSpec((1, C1), lambda i: (0, 0)),
            pl.BlockSpec((1, C1), lambda i: (0, 0)),
            pl.BlockSpec((1, C1), lambda i: (0, 0)),
            pl.BlockSpec((1, C1), lambda i: (0, 0)),
            pl.BlockSpec((C1, C2), lambda i: (0, 0)),
            pl.BlockSpec((1, C2), lambda i: (0, 0)),
        ],
        out_specs=[
            pl.BlockSpec((M, C2), lambda i: (i, 0)),
            pl.BlockSpec((1, C2), lambda i: (0, 0)),
            pl.BlockSpec((1, C2), lambda i: (0, 0)),
        ],
        out_shape=[
            jax.ShapeDtypeStruct((R, C2), jnp.float32),
            jax.ShapeDtypeStruct((1, C2), jnp.float32),
            jax.ShapeDtypeStruct((1, C2), jnp.float32),
        ],
    )(y1, s1, q1, row2(g1), row2(be1), w2T, row2(b2))

    out_flat = pl.pallas_call(
        functools.partial(_final_body, R),
        grid=(G,),
        in_specs=[
            pl.BlockSpec((M, C2), lambda i: (i, 0)),
            pl.BlockSpec((1, C2), lambda i: (0, 0)),
            pl.BlockSpec((1, C2), lambda i: (0, 0)),
            pl.BlockSpec((1, C2), lambda i: (0, 0)),
            pl.BlockSpec((1, C2), lambda i: (0, 0)),
        ],
        out_specs=pl.BlockSpec((M, C2), lambda i: (i, 0)),
        out_shape=jax.ShapeDtypeStruct((R, C2), jnp.float32),
    )(y2, s2, q2, row2(g2), row2(be2))

    return jnp.transpose(out_flat.reshape(B, N, C2), (0, 2, 1))
